# baseline (speedup 1.0000x reference)
import jax
import jax.numpy as jnp
from jax.experimental import pallas as pl

N = 50000
HEADS = 8
HID = 8
NUM_CLASSES = 121


def _elu_kernel(x_ref, o_ref):
    v = x_ref[...]
    o_ref[...] = jnp.where(v > 0, v, jnp.exp(jnp.minimum(v, 0.0)) - 1.0)


def _pallas_elu(x):
    n, c = x.shape
    pad = (-n) % 8
    xp = jnp.pad(x, ((0, pad), (0, 0)))
    out = pl.pallas_call(
        _elu_kernel,
        out_shape=jax.ShapeDtypeStruct(xp.shape, xp.dtype),
    )(xp)
    return out[:n]


def _gat(x, edge_index, W, att_src, att_dst, bias, heads, out_ch, concat):
    n = x.shape[0]
    h = (x @ W).reshape(n, heads, out_ch)
    a_src = (h * att_src).sum(-1)
    a_dst = (h * att_dst).sum(-1)
    src = edge_index[0]
    dst = edge_index[1]
    alpha = a_src[src] + a_dst[dst]
    alpha = jax.nn.leaky_relu(alpha, 0.2)
    amax = jax.ops.segment_max(alpha, dst, num_segments=n)
    amax = jnp.where(jnp.isfinite(amax), amax, 0.0)
    alpha = jnp.exp(alpha - amax[dst])
    denom = jax.ops.segment_sum(alpha, dst, num_segments=n)
    alpha = alpha / (denom[dst] + 1e-16)
    msg = h[src] * alpha[:, :, None]
    out = jax.ops.segment_sum(msg, dst, num_segments=n)
    if concat:
        out = out.reshape(n, heads * out_ch)
    else:
        out = out.mean(axis=1)
    return out + bias


def kernel(x, edge_index, W1, a_src1, a_dst1, b1, W2, a_src2, a_dst2, b2, W3, a_src3, a_dst3, b3):
    x1 = _gat(x, edge_index, W1, a_src1, a_dst1, b1, HEADS, HID, True)
    x = _pallas_elu(x1)
    x1 = _gat(x, edge_index, W2, a_src2, a_dst2, b2, 1, HEADS * HID, True)
    x = _pallas_elu(x1)
    x1 = _gat(x, edge_index, W3, a_src3, a_dst3, b3, 1, NUM_CLASSES, False)
    return x1


# speedup vs baseline: 29.6769x; 29.6769x over previous
import functools

import jax
import jax.numpy as jnp
from jax import lax
from jax.experimental import pallas as pl
from jax.experimental.pallas import tpu as pltpu
from jax.experimental.pallas import tpu_sc as plsc

N_NODES = 50000
N_EDGES = 800000
NPAD = 51200
ROWS_PER_TILE = NPAD // 16
K = 256
NCHUNK_A = 98
NCHUNK_B = 196
EPT_A = NCHUNK_A * K
EPT_B = NCHUNK_B * K
EPAD = EPT_B * 16
DUMP_ROW = N_NODES
BLK = 256
GRID = NPAD // BLK

_SC_PARAMS = pltpu.CompilerParams(
    needs_layout_passes=False, use_tc_tiling_on_sc=False)


def _elu(v):
    return jnp.where(v > 0, v, jnp.exp(jnp.minimum(v, 0.0)) - 1.0)


def _mesh():
    return plsc.VectorSubcoreMesh(core_axis_name="c", subcore_axis_name="s")


def _make_w_kernel(l1_mode):
    @functools.partial(
        pl.kernel,
        out_type=(jax.ShapeDtypeStruct((EPAD, 8), jnp.float32),
                  jax.ShapeDtypeStruct((2, NPAD, 8), jnp.float32)),
        mesh=_mesh(),
        compiler_params=_SC_PARAMS,
        scratch_types=[
            pltpu.VMEM((2, 128), jnp.int32),
            pltpu.VMEM((2, 128), jnp.int32),
            pltpu.VMEM((2, 128, 16), jnp.float32),
            pltpu.VMEM((2, 128, 16), jnp.float32),
            pltpu.VMEM((2, 128, 8), jnp.float32),
            pltpu.SemaphoreType.DMA,
            pltpu.SemaphoreType.DMA,
            pltpu.VMEM_SHARED((NPAD, 8), jnp.float32),
        ],
    )
    def w_kernel(src_hbm, dst_hbm, atbl_hbm, w_hbm, den_hbm,
                 src_i, dst_i, asb, adb, wb, sem0, sem1, den_sp):
        c = lax.axis_index("c")
        s = lax.axis_index("s")
        wid = s * 2 + c
        lane = lax.iota(jnp.int32, 16)
        zf = jnp.zeros((16,), jnp.float32)

        for q in range(2):
            iq = jnp.full((16,), q, jnp.int32)

            @pl.loop(0, 64)
            def _(t):
                r = t * 2 + lax.shift_right_logical(lane, 3)
                plsc.store_scatter(wb, [iq, r, lane & 7], zf)

        row0 = s * ROWS_PER_TILE

        @pl.loop(0, 25)
        def _(t):
            pltpu.sync_copy(wb.at[0], den_sp.at[pl.ds(row0 + t * 128, 128)])

        plsc.subcore_barrier()

        ebase = wid * (EPT_A // 128)
        wbase = wid * EPT_A

        @pl.loop(0, NCHUNK_A)
        def _(j):
            r0 = ebase + j * 2
            pltpu.sync_copy(src_hbm.at[pl.ds(r0, 2)], src_i)
            pltpu.sync_copy(dst_hbm.at[pl.ds(r0, 2)], dst_i)
            copies = []
            for q in range(2):
                copies.append(pltpu.async_copy(
                    atbl_hbm.at[src_i.at[q]], asb.at[q], sem0))
                copies.append(pltpu.async_copy(
                    atbl_hbm.at[dst_i.at[q]], adb.at[q], sem1))
            for cp in copies:
                cp.wait()

            for q in range(2):
                iq = jnp.full((16,), q, jnp.int32)
                if l1_mode:
                    @pl.loop(0, 64)
                    def _(t):
                        r = t * 2 + lax.shift_right_logical(lane, 3)
                        hh = lane & 7
                        a = (plsc.load_gather(asb, [iq, r, hh])
                             + plsc.load_gather(adb, [iq, r, 8 + hh]))
                        w = jnp.exp(jnp.maximum(a, 0.2 * a))
                        plsc.store_scatter(wb, [iq, r, hh], w)
                else:
                    zi = jnp.zeros((16,), jnp.int32)

                    @pl.loop(0, 8)
                    def _(t):
                        r = t * 16 + lane
                        a = (plsc.load_gather(asb, [iq, r, zi])
                             + plsc.load_gather(adb, [iq, r, zi + 1]))
                        w = jnp.exp(jnp.maximum(a, 0.2 * a))
                        plsc.store_scatter(wb, [iq, r, zi], w)

            for q in range(2):
                pltpu.sync_copy(wb.at[q], den_sp.at[dst_i.at[q]], add=True)
                pltpu.sync_copy(
                    wb.at[q],
                    w_hbm.at[pl.ds(wbase + j * K + q * 128, 128), :])

        plsc.subcore_barrier()

        @pl.loop(0, 25)
        def _(t):
            r = row0 + t * 128
            pltpu.sync_copy(den_sp.at[pl.ds(r, 128)],
                            den_hbm.at[c, pl.ds(r, 128), :])

    return w_kernel


def _make_msg_kernel(l1_mode):
    @functools.partial(
        pl.kernel,
        out_type=jax.ShapeDtypeStruct((2, NPAD, 32), jnp.float32),
        mesh=_mesh(),
        compiler_params=_SC_PARAMS,
        scratch_types=[
            pltpu.VMEM((2, 128), jnp.int32),
            pltpu.VMEM((2, 128), jnp.int32),
            pltpu.VMEM((256, 8), jnp.float32),
            pltpu.VMEM((2, 128, 64), jnp.float32),
            pltpu.VMEM((2, 128, 32), jnp.float32),
            pltpu.SemaphoreType.DMA,
            pltpu.VMEM_SHARED((NPAD, 32), jnp.float32),
        ],
    )
    def msg_kernel(src_hbm, dst_hbm, w_hbm, feat_hbm, out_hbm,
                   src_i, dst_i, wb, fb, msgb, sem0, out_sp):
        c = lax.axis_index("c")
        s = lax.axis_index("s")
        lane = lax.iota(jnp.int32, 16)
        zi = jnp.zeros((16,), jnp.int32)
        zf = jnp.zeros((16,), jnp.float32)

        for q in range(2):
            iq = jnp.full((16,), q, jnp.int32)

            @pl.loop(0, 128)
            def _(e):
                ie = zi + e
                plsc.store_scatter(msgb, [iq, ie, lane], zf)
                plsc.store_scatter(msgb, [iq, ie, lane + 16], zf)

        row0 = s * ROWS_PER_TILE

        @pl.loop(0, 25)
        def _(t):
            pltpu.sync_copy(msgb.at[0], out_sp.at[pl.ds(row0 + t * 128, 128)])

        plsc.subcore_barrier()

        ebase = s * (EPT_B // 128)
        wbase = s * EPT_B

        @pl.loop(0, NCHUNK_B)
        def _(j):
            r0 = ebase + j * 2
            pltpu.sync_copy(src_hbm.at[pl.ds(r0, 2)], src_i)
            pltpu.sync_copy(dst_hbm.at[pl.ds(r0, 2)], dst_i)
            pltpu.sync_copy(w_hbm.at[pl.ds(wbase + j * K, K), :], wb)
            copies = []
            for q in range(2):
                copies.append(pltpu.async_copy(
                    feat_hbm.at[src_i.at[q]], fb.at[q], sem0))
            for cp in copies:
                cp.wait()

            fcol0 = c * 32 + lane
            if l1_mode:
                wcol0 = c * 4 + lax.shift_right_logical(lane, 3)
                wcol1 = wcol0 + 2
            else:
                wcol0 = zi
                wcol1 = zi

            for q in range(2):
                iq = jnp.full((16,), q, jnp.int32)

                @pl.loop(0, 128)
                def _(e):
                    ie = zi + e
                    iw = ie + q * 128
                    w0 = plsc.load_gather(wb, [iw, wcol0])
                    f0 = plsc.load_gather(fb, [iq, ie, fcol0])
                    plsc.store_scatter(msgb, [iq, ie, lane], w0 * f0)
                    w1 = plsc.load_gather(wb, [iw, wcol1])
                    f1 = plsc.load_gather(fb, [iq, ie, fcol0 + 16])
                    plsc.store_scatter(msgb, [iq, ie, lane + 16], w1 * f1)

            for q in range(2):
                pltpu.sync_copy(msgb.at[q], out_sp.at[dst_i.at[q]], add=True)

        plsc.subcore_barrier()

        @pl.loop(0, 25)
        def _(t):
            r = row0 + t * 128
            pltpu.sync_copy(out_sp.at[pl.ds(r, 128)],
                            out_hbm.at[c, pl.ds(r, 128), :])

    return msg_kernel


_w_l1 = _make_w_kernel(True)
_w_single = _make_w_kernel(False)
_msg_l1 = _make_msg_kernel(True)
_msg_single = _make_msg_kernel(False)


def _tc1_body(x_ref, w1_ref, as_ref, ad_ref, h_ref, atbl_ref):
    h = jnp.dot(x_ref[...], w1_ref[...], preferred_element_type=jnp.float32)
    a_s = jnp.dot(h, as_ref[...], preferred_element_type=jnp.float32)
    a_d = jnp.dot(h, ad_ref[...], preferred_element_type=jnp.float32)
    h_ref[...] = h
    atbl_ref[...] = jnp.concatenate([a_s, a_d], axis=1)


def _tc2_body(p0_ref, p1_ref, d0_ref, d1_ref, rep_ref, b_ref, w_ref, m_ref,
              feat_ref, atbl_ref):
    rep = rep_ref[...]
    den = d0_ref[...] + d1_ref[...] + 1e-16
    dd0 = jnp.dot(den[:, 0:4], rep, preferred_element_type=jnp.float32)
    dd1 = jnp.dot(den[:, 4:8], rep, preferred_element_type=jnp.float32)
    x = jnp.concatenate([p0_ref[...] / dd0, p1_ref[...] / dd1],
                        axis=1) + b_ref[...]
    x = _elu(x)
    feat_ref[...] = jnp.dot(x, w_ref[...], preferred_element_type=jnp.float32)
    atbl_ref[...] = jnp.dot(x, m_ref[...], preferred_element_type=jnp.float32)


def _tc3_body(p0_ref, p1_ref, d0_ref, d1_ref, b_ref, m_ref, feat_ref,
              atbl_ref):
    d = d0_ref[...][:, 0:1] + d1_ref[...][:, 0:1] + 1e-16
    x = jnp.concatenate([p0_ref[...] / d, p1_ref[...] / d], axis=1) + b_ref[...]
    x = _elu(x)
    feat_ref[...] = x
    atbl_ref[...] = jnp.dot(x, m_ref[...], preferred_element_type=jnp.float32)


def _tc4_body(p0_ref, p1_ref, d0_ref, d1_ref, w_ref, b_ref, out_ref):
    d = d0_ref[...][:, 0:1] + d1_ref[...][:, 0:1] + 1e-16
    agg = jnp.concatenate([p0_ref[...] / d, p1_ref[...] / d], axis=1)
    out_ref[...] = jnp.dot(agg, w_ref[...],
                           preferred_element_type=jnp.float32) + b_ref[...]


def _row_spec(c):
    return pl.BlockSpec((BLK, c), lambda i: (i, 0))


def _full_spec(r, c):
    return pl.BlockSpec((r, c), lambda i: (0, 0))


_tc1 = pl.pallas_call(
    _tc1_body,
    grid=(GRID,),
    in_specs=[_row_spec(128), _full_spec(128, 64), _full_spec(64, 8),
              _full_spec(64, 8)],
    out_specs=[_row_spec(64), _row_spec(16)],
    out_shape=[jax.ShapeDtypeStruct((NPAD, 64), jnp.float32),
               jax.ShapeDtypeStruct((NPAD, 16), jnp.float32)],
)

_tc2 = pl.pallas_call(
    _tc2_body,
    grid=(GRID,),
    in_specs=[_row_spec(32), _row_spec(32), _row_spec(8), _row_spec(8),
              _full_spec(4, 32), _full_spec(1, 64), _full_spec(64, 64),
              _full_spec(64, 16)],
    out_specs=[_row_spec(64), _row_spec(16)],
    out_shape=[jax.ShapeDtypeStruct((NPAD, 64), jnp.float32),
               jax.ShapeDtypeStruct((NPAD, 16), jnp.float32)],
)

_tc3 = pl.pallas_call(
    _tc3_body,
    grid=(GRID,),
    in_specs=[_row_spec(32), _row_spec(32), _row_spec(8), _row_spec(8),
              _full_spec(1, 64), _full_spec(64, 16)],
    out_specs=[_row_spec(64), _row_spec(16)],
    out_shape=[jax.ShapeDtypeStruct((NPAD, 64), jnp.float32),
               jax.ShapeDtypeStruct((NPAD, 16), jnp.float32)],
)

_tc4 = pl.pallas_call(
    _tc4_body,
    grid=(GRID,),
    in_specs=[_row_spec(32), _row_spec(32), _row_spec(8), _row_spec(8),
              _full_spec(64, 128), _full_spec(1, 128)],
    out_specs=_row_spec(128),
    out_shape=jax.ShapeDtypeStruct((NPAD, 128), jnp.float32),
)


def kernel(x, edge_index, W1, a_src1, a_dst1, b1, W2, a_src2, a_dst2, b2,
           W3, a_src3, a_dst3, b3):
    f32 = jnp.float32
    xp = jnp.pad(x.astype(f32), ((0, NPAD - N_NODES), (0, 128 - x.shape[1])))
    w1p = jnp.pad(W1, ((0, 128 - W1.shape[0]), (0, 0)))
    eye8 = jnp.eye(8, dtype=f32)
    as1 = (a_src1[0][:, :, None] * eye8[:, None, :]).reshape(64, 8)
    ad1 = (a_dst1[0][:, :, None] * eye8[:, None, :]).reshape(64, 8)
    m2 = jnp.zeros((64, 16), f32)
    m2 = m2.at[:, 0].set(W2 @ a_src2[0, 0]).at[:, 1].set(W2 @ a_dst2[0, 0])
    m3 = jnp.zeros((64, 16), f32)
    m3 = m3.at[:, 0].set(W3 @ a_src3[0, 0]).at[:, 1].set(W3 @ a_dst3[0, 0])
    rep = jnp.repeat(jnp.eye(4, dtype=f32), 8, axis=1)
    w3p = jnp.pad(W3, ((0, 0), (0, 128 - W3.shape[1])))
    b3p = jnp.pad(b3, (0, 128 - b3.shape[0])).reshape(1, 128)
    b1r = b1.reshape(1, 64)
    b2r = b2.reshape(1, 64)
    src = jnp.concatenate(
        [edge_index[0], jnp.zeros((EPAD - N_EDGES,), jnp.int32)])
    dst = jnp.concatenate(
        [edge_index[1], jnp.full((EPAD - N_EDGES,), DUMP_ROW, jnp.int32)])
    src2d = src.reshape(EPAD // 128, 128)
    dst2d = dst.reshape(EPAD // 128, 128)

    h1, atbl1 = _tc1(xp, w1p, as1, ad1)
    w1e, den1 = _w_l1(src2d, dst2d, atbl1)
    out1 = _msg_l1(src2d, dst2d, w1e, h1)
    feat2, atbl2 = _tc2(out1[0], out1[1], den1[0], den1[1], rep, b1r, W2, m2)
    w2e, den2 = _w_single(src2d, dst2d, atbl2)
    out2 = _msg_single(src2d, dst2d, w2e, feat2)
    feat3, atbl3 = _tc3(out2[0], out2[1], den2[0], den2[1], b2r, m3)
    w3e, den3 = _w_single(src2d, dst2d, atbl3)
    out3 = _msg_single(src2d, dst2d, w3e, feat3)
    res = _tc4(out3[0], out3[1], den3[0], den3[1], w3p, b3p)
    return res[:N_NODES, :121]


# speedup vs baseline: 39.7364x; 1.3390x over previous
import functools

import jax
import jax.numpy as jnp
from jax import lax
from jax.experimental import pallas as pl
from jax.experimental.pallas import tpu as pltpu
from jax.experimental.pallas import tpu_sc as plsc

N_NODES = 50000
N_EDGES = 800000
NPAD = 50176
NROWBLK = NPAD // 128
K = 256
NCHUNK_A = 98
NCHUNK_B = 196
EPT_A = NCHUNK_A * K
EPT_B = NCHUNK_B * K
EPAD = EPT_B * 16
DUMP_ROW = N_NODES
BLK = 256
GRID = NPAD // BLK

_SC_PARAMS = pltpu.CompilerParams(
    needs_layout_passes=False, use_tc_tiling_on_sc=False)


def _elu(v):
    return jnp.where(v > 0, v, jnp.exp(jnp.minimum(v, 0.0)) - 1.0)


def _mesh():
    return plsc.VectorSubcoreMesh(core_axis_name="c", subcore_axis_name="s")


def _make_w_kernel(l1_mode):
    @functools.partial(
        pl.kernel,
        out_type=(jax.ShapeDtypeStruct((EPAD, 8), jnp.float32),
                  jax.ShapeDtypeStruct((2, NPAD, 8), jnp.float32)),
        mesh=_mesh(),
        compiler_params=_SC_PARAMS,
        scratch_types=[
            pltpu.VMEM((2, 2, 128), jnp.int32),
            pltpu.VMEM((2, 2, 128), jnp.int32),
            pltpu.VMEM((2, 2, 128, 16), jnp.float32),
            pltpu.VMEM((2, 2, 128, 16), jnp.float32),
            pltpu.VMEM((256, 8), jnp.float32),
            pltpu.SemaphoreType.DMA,
            pltpu.VMEM_SHARED((NPAD, 8), jnp.float32),
        ],
    )
    def w_kernel(src_hbm, dst_hbm, atbl_hbm, w_hbm, den_hbm,
                 src_i, dst_i, asb, adb, wb, sem_g, den_sp):
        c = lax.axis_index("c")
        s = lax.axis_index("s")
        wid = s * 2 + c
        lane = lax.iota(jnp.int32, 16)
        zf = jnp.zeros((16,), jnp.float32)

        @pl.loop(0, 128)
        def _(t):
            r = t * 2 + lax.shift_right_logical(lane, 3)
            plsc.store_scatter(wb, [r, lane & 7], zf)

        @pl.loop(0, 25)
        def _(t):
            g = t * 16 + s

            @pl.when(g < NROWBLK)
            def _():
                pltpu.sync_copy(wb.at[pl.ds(0, 128)],
                                den_sp.at[pl.ds(g * 128, 128)])

        plsc.subcore_barrier()

        ebase = wid * (EPT_A // 128)
        wbase = wid * EPT_A

        def issue(jj, slot):
            r0 = ebase + jj * 2
            pltpu.sync_copy(src_hbm.at[pl.ds(r0, 2)], src_i.at[slot])
            pltpu.sync_copy(dst_hbm.at[pl.ds(r0, 2)], dst_i.at[slot])
            for q in range(2):
                pltpu.async_copy(
                    atbl_hbm.at[src_i.at[slot, q]], asb.at[slot, q], sem_g)
                pltpu.async_copy(
                    atbl_hbm.at[dst_i.at[slot, q]], adb.at[slot, q], sem_g)

        def drain(slot):
            for q in range(2):
                pltpu.make_async_copy(
                    atbl_hbm.at[pl.ds(0, 128)], asb.at[slot, q], sem_g).wait()
                pltpu.make_async_copy(
                    atbl_hbm.at[pl.ds(0, 128)], adb.at[slot, q], sem_g).wait()

        issue(0, 0)

        @pl.loop(0, NCHUNK_A, step=2)
        def _(j):
            for slot in range(2):
                jj = j + slot
                issue(jnp.minimum(jj + 1, NCHUNK_A - 1), 1 - slot)
                drain(slot)
                for q in range(2):
                    iq = jnp.full((16,), q, jnp.int32)
                    if l1_mode:
                        @pl.loop(0, 64)
                        def _(t):
                            r = t * 2 + lax.shift_right_logical(lane, 3)
                            hh = lane & 7
                            a = (plsc.load_gather(asb.at[slot], [iq, r, hh])
                                 + plsc.load_gather(adb.at[slot],
                                                    [iq, r, 8 + hh]))
                            w = jnp.exp(jnp.maximum(a, 0.2 * a))
                            plsc.store_scatter(wb, [q * 128 + r, hh], w)
                    else:
                        zi = jnp.zeros((16,), jnp.int32)

                        @pl.loop(0, 8)
                        def _(t):
                            r = t * 16 + lane
                            a = (plsc.load_gather(asb.at[slot], [iq, r, zi])
                                 + plsc.load_gather(adb.at[slot],
                                                    [iq, r, zi + 1]))
                            w = jnp.exp(jnp.maximum(a, 0.2 * a))
                            plsc.store_scatter(wb, [q * 128 + r, zi], w)

                for q in range(2):
                    pltpu.sync_copy(wb.at[pl.ds(q * 128, 128)],
                                    den_sp.at[dst_i.at[slot, q]], add=True)
                pltpu.sync_copy(
                    wb, w_hbm.at[pl.ds(wbase + jj * K, K), :])

        drain(0)
        plsc.subcore_barrier()

        @pl.loop(0, 25)
        def _(t):
            g = t * 16 + s

            @pl.when(g < NROWBLK)
            def _():
                r = g * 128
                pltpu.sync_copy(den_sp.at[pl.ds(r, 128)],
                                den_hbm.at[c, pl.ds(r, 128), :])

    return w_kernel


def _make_msg_kernel(l1_mode):
    @functools.partial(
        pl.kernel,
        out_type=jax.ShapeDtypeStruct((2, NPAD, 32), jnp.float32),
        mesh=_mesh(),
        compiler_params=_SC_PARAMS,
        scratch_types=[
            pltpu.VMEM((2, 2, 128), jnp.int32),
            pltpu.VMEM((2, 2, 128), jnp.int32),
            pltpu.VMEM((2, 256, 8), jnp.float32),
            pltpu.VMEM((2, 2, 128, 32), jnp.float32),
            pltpu.VMEM((2, 128, 32), jnp.float32),
            pltpu.SemaphoreType.DMA,
            pltpu.SemaphoreType.DMA,
            pltpu.VMEM_SHARED((NPAD, 32), jnp.float32),
        ],
    )
    def msg_kernel(src_hbm, dst_hbm, w_hbm, flo_hbm, fhi_hbm, out_hbm,
                   src_i, dst_i, wb, fb, msgb, sem_g, sem_w, out_sp):
        c = lax.axis_index("c")
        s = lax.axis_index("s")
        lane = lax.iota(jnp.int32, 16)
        zi = jnp.zeros((16,), jnp.int32)
        zf = jnp.zeros((16,), jnp.float32)

        for q in range(2):
            iq = jnp.full((16,), q, jnp.int32)

            @pl.loop(0, 128)
            def _(e):
                ie = zi + e
                plsc.store_scatter(msgb, [iq, ie, lane], zf)
                plsc.store_scatter(msgb, [iq, ie, lane + 16], zf)

        @pl.loop(0, 25)
        def _(t):
            g = t * 16 + s

            @pl.when(g < NROWBLK)
            def _():
                pltpu.sync_copy(msgb.at[0], out_sp.at[pl.ds(g * 128, 128)])

        plsc.subcore_barrier()

        ebase = s * (EPT_B // 128)
        wbase = s * EPT_B

        def issue(jj, slot):
            r0 = ebase + jj * 2
            pltpu.sync_copy(src_hbm.at[pl.ds(r0, 2)], src_i.at[slot])
            pltpu.sync_copy(dst_hbm.at[pl.ds(r0, 2)], dst_i.at[slot])
            pltpu.async_copy(
                w_hbm.at[pl.ds(wbase + jj * K, K), :], wb.at[slot], sem_w)
            for q in range(2):
                @pl.when(c == 0)
                def _():
                    pltpu.async_copy(
                        flo_hbm.at[src_i.at[slot, q]], fb.at[slot, q], sem_g)

                @pl.when(c == 1)
                def _():
                    pltpu.async_copy(
                        fhi_hbm.at[src_i.at[slot, q]], fb.at[slot, q], sem_g)

        def drain(slot):
            pltpu.make_async_copy(
                w_hbm.at[pl.ds(0, K), :], wb.at[slot], sem_w).wait()
            for q in range(2):
                pltpu.make_async_copy(
                    flo_hbm.at[pl.ds(0, 128)], fb.at[slot, q], sem_g).wait()

        issue(0, 0)

        if l1_mode:
            wcol0 = c * 4 + lax.shift_right_logical(lane, 3)
            wcol1 = wcol0 + 2
        else:
            wcol0 = zi
            wcol1 = zi

        @pl.loop(0, NCHUNK_B, step=2)
        def _(j):
            for slot in range(2):
                jj = j + slot
                issue(jnp.minimum(jj + 1, NCHUNK_B - 1), 1 - slot)
                drain(slot)

                for q in range(2):
                    iq = jnp.full((16,), q, jnp.int32)

                    @pl.loop(0, 128)
                    def _(e):
                        ie = zi + e
                        iw = ie + q * 128
                        w0 = plsc.load_gather(wb.at[slot], [iw, wcol0])
                        f0 = plsc.load_gather(fb.at[slot], [iq, ie, lane])
                        plsc.store_scatter(msgb, [iq, ie, lane], w0 * f0)
                        w1 = plsc.load_gather(wb.at[slot], [iw, wcol1])
                        f1 = plsc.load_gather(fb.at[slot], [iq, ie, lane + 16])
                        plsc.store_scatter(msgb, [iq, ie, lane + 16], w1 * f1)

                for q in range(2):
                    pltpu.sync_copy(msgb.at[q],
                                    out_sp.at[dst_i.at[slot, q]], add=True)

        drain(0)
        plsc.subcore_barrier()

        @pl.loop(0, 25)
        def _(t):
            g = t * 16 + s

            @pl.when(g < NROWBLK)
            def _():
                r = g * 128
                pltpu.sync_copy(out_sp.at[pl.ds(r, 128)],
                                out_hbm.at[c, pl.ds(r, 128), :])

    return msg_kernel


_w_l1 = _make_w_kernel(True)
_w_single = _make_w_kernel(False)
_msg_l1 = _make_msg_kernel(True)
_msg_single = _make_msg_kernel(False)


def _tc1_body(x_ref, w1_ref, as_ref, ad_ref, hlo_ref, hhi_ref, atbl_ref):
    h = jnp.dot(x_ref[...], w1_ref[...], preferred_element_type=jnp.float32)
    a_s = jnp.dot(h, as_ref[...], preferred_element_type=jnp.float32)
    a_d = jnp.dot(h, ad_ref[...], preferred_element_type=jnp.float32)
    hlo_ref[...] = h[:, 0:32]
    hhi_ref[...] = h[:, 32:64]
    atbl_ref[...] = jnp.concatenate([a_s, a_d], axis=1)


def _tc2_body(p0_ref, p1_ref, d0_ref, d1_ref, rep_ref, b_ref, w_ref, m_ref,
              flo_ref, fhi_ref, atbl_ref):
    rep = rep_ref[...]
    den = d0_ref[...] + d1_ref[...] + 1e-16
    dd0 = jnp.dot(den[:, 0:4], rep, preferred_element_type=jnp.float32)
    dd1 = jnp.dot(den[:, 4:8], rep, preferred_element_type=jnp.float32)
    x = jnp.concatenate([p0_ref[...] / dd0, p1_ref[...] / dd1],
                        axis=1) + b_ref[...]
    x = _elu(x)
    h = jnp.dot(x, w_ref[...], preferred_element_type=jnp.float32)
    flo_ref[...] = h[:, 0:32]
    fhi_ref[...] = h[:, 32:64]
    atbl_ref[...] = jnp.dot(x, m_ref[...], preferred_element_type=jnp.float32)


def _tc3_body(p0_ref, p1_ref, d0_ref, d1_ref, b_ref, m_ref, flo_ref, fhi_ref,
              atbl_ref):
    d = d0_ref[...][:, 0:1] + d1_ref[...][:, 0:1] + 1e-16
    x = jnp.concatenate([p0_ref[...] / d, p1_ref[...] / d], axis=1) + b_ref[...]
    x = _elu(x)
    flo_ref[...] = x[:, 0:32]
    fhi_ref[...] = x[:, 32:64]
    atbl_ref[...] = jnp.dot(x, m_ref[...], preferred_element_type=jnp.float32)


def _tc4_body(p0_ref, p1_ref, d0_ref, d1_ref, w_ref, b_ref, out_ref):
    d = d0_ref[...][:, 0:1] + d1_ref[...][:, 0:1] + 1e-16
    agg = jnp.concatenate([p0_ref[...] / d, p1_ref[...] / d], axis=1)
    out_ref[...] = jnp.dot(agg, w_ref[...],
                           preferred_element_type=jnp.float32) + b_ref[...]


def _row_spec(c):
    return pl.BlockSpec((BLK, c), lambda i: (i, 0))


def _full_spec(r, c):
    return pl.BlockSpec((r, c), lambda i: (0, 0))


_tc1 = pl.pallas_call(
    _tc1_body,
    grid=(GRID,),
    in_specs=[_row_spec(128), _full_spec(128, 64), _full_spec(64, 8),
              _full_spec(64, 8)],
    out_specs=[_row_spec(32), _row_spec(32), _row_spec(16)],
    out_shape=[jax.ShapeDtypeStruct((NPAD, 32), jnp.float32),
               jax.ShapeDtypeStruct((NPAD, 32), jnp.float32),
               jax.ShapeDtypeStruct((NPAD, 16), jnp.float32)],
)

_tc2 = pl.pallas_call(
    _tc2_body,
    grid=(GRID,),
    in_specs=[_row_spec(32), _row_spec(32), _row_spec(8), _row_spec(8),
              _full_spec(4, 32), _full_spec(1, 64), _full_spec(64, 64),
              _full_spec(64, 16)],
    out_specs=[_row_spec(32), _row_spec(32), _row_spec(16)],
    out_shape=[jax.ShapeDtypeStruct((NPAD, 32), jnp.float32),
               jax.ShapeDtypeStruct((NPAD, 32), jnp.float32),
               jax.ShapeDtypeStruct((NPAD, 16), jnp.float32)],
)

_tc3 = pl.pallas_call(
    _tc3_body,
    grid=(GRID,),
    in_specs=[_row_spec(32), _row_spec(32), _row_spec(8), _row_spec(8),
              _full_spec(1, 64), _full_spec(64, 16)],
    out_specs=[_row_spec(32), _row_spec(32), _row_spec(16)],
    out_shape=[jax.ShapeDtypeStruct((NPAD, 32), jnp.float32),
               jax.ShapeDtypeStruct((NPAD, 32), jnp.float32),
               jax.ShapeDtypeStruct((NPAD, 16), jnp.float32)],
)

_tc4 = pl.pallas_call(
    _tc4_body,
    grid=(GRID,),
    in_specs=[_row_spec(32), _row_spec(32), _row_spec(8), _row_spec(8),
              _full_spec(64, 128), _full_spec(1, 128)],
    out_specs=_row_spec(128),
    out_shape=jax.ShapeDtypeStruct((NPAD, 128), jnp.float32),
)


def kernel(x, edge_index, W1, a_src1, a_dst1, b1, W2, a_src2, a_dst2, b2,
           W3, a_src3, a_dst3, b3):
    f32 = jnp.float32
    xp = jnp.pad(x.astype(f32), ((0, NPAD - N_NODES), (0, 128 - x.shape[1])))
    w1p = jnp.pad(W1, ((0, 128 - W1.shape[0]), (0, 0)))
    eye8 = jnp.eye(8, dtype=f32)
    as1 = (a_src1[0][:, :, None] * eye8[:, None, :]).reshape(64, 8)
    ad1 = (a_dst1[0][:, :, None] * eye8[:, None, :]).reshape(64, 8)
    m2 = jnp.zeros((64, 16), f32)
    m2 = m2.at[:, 0].set(W2 @ a_src2[0, 0]).at[:, 1].set(W2 @ a_dst2[0, 0])
    m3 = jnp.zeros((64, 16), f32)
    m3 = m3.at[:, 0].set(W3 @ a_src3[0, 0]).at[:, 1].set(W3 @ a_dst3[0, 0])
    rep = jnp.repeat(jnp.eye(4, dtype=f32), 8, axis=1)
    w3p = jnp.pad(W3, ((0, 0), (0, 128 - W3.shape[1])))
    b3p = jnp.pad(b3, (0, 128 - b3.shape[0])).reshape(1, 128)
    b1r = b1.reshape(1, 64)
    b2r = b2.reshape(1, 64)
    src = jnp.concatenate(
        [edge_index[0], jnp.zeros((EPAD - N_EDGES,), jnp.int32)])
    dst = jnp.concatenate(
        [edge_index[1], jnp.full((EPAD - N_EDGES,), DUMP_ROW, jnp.int32)])
    src2d = src.reshape(EPAD // 128, 128)
    dst2d = dst.reshape(EPAD // 128, 128)

    h1lo, h1hi, atbl1 = _tc1(xp, w1p, as1, ad1)
    w1e, den1 = _w_l1(src2d, dst2d, atbl1)
    out1 = _msg_l1(src2d, dst2d, w1e, h1lo, h1hi)
    f2lo, f2hi, atbl2 = _tc2(out1[0], out1[1], den1[0], den1[1], rep, b1r,
                             W2, m2)
    w2e, den2 = _w_single(src2d, dst2d, atbl2)
    out2 = _msg_single(src2d, dst2d, w2e, f2lo, f2hi)
    f3lo, f3hi, atbl3 = _tc3(out2[0], out2[1], den2[0], den2[1], b2r, m3)
    w3e, den3 = _w_single(src2d, dst2d, atbl3)
    out3 = _msg_single(src2d, dst2d, w3e, f3lo, f3hi)
    res = _tc4(out3[0], out3[1], den3[0], den3[1], w3p, b3p)
    return res[:N_NODES, :121]


# speedup vs baseline: 44.2266x; 1.1130x over previous
import functools

import jax
import jax.numpy as jnp
from jax import lax
from jax.experimental import pallas as pl
from jax.experimental.pallas import tpu as pltpu
from jax.experimental.pallas import tpu_sc as plsc

N_NODES = 50000
N_EDGES = 800000
NPAD = 50176
NROWBLK = NPAD // 128
K = 256
NCHUNK_A = 98
NCHUNK_B = 196
EPT_A = NCHUNK_A * K
EPT_B = NCHUNK_B * K
EPAD = EPT_B * 16
DUMP_ROW = N_NODES
BLK = 256
GRID = NPAD // BLK

_SC_PARAMS = pltpu.CompilerParams(
    needs_layout_passes=False, use_tc_tiling_on_sc=False)


def _elu(v):
    return jnp.where(v > 0, v, jnp.exp(jnp.minimum(v, 0.0)) - 1.0)


def _mesh():
    return plsc.VectorSubcoreMesh(core_axis_name="c", subcore_axis_name="s")


def _make_w_kernel(l1_mode):
    @functools.partial(
        pl.kernel,
        out_type=(jax.ShapeDtypeStruct((EPAD, 8), jnp.float32),
                  jax.ShapeDtypeStruct((2, NPAD, 8), jnp.float32)),
        mesh=_mesh(),
        compiler_params=_SC_PARAMS,
        scratch_types=[
            pltpu.VMEM((2, 2, 2, 128), jnp.int32),
            pltpu.VMEM((2, 2, 128, 16), jnp.float32),
            pltpu.VMEM((2, 2, 128, 16), jnp.float32),
            pltpu.VMEM((256, 8), jnp.float32),
            pltpu.SemaphoreType.DMA,
            pltpu.VMEM_SHARED((NPAD, 8), jnp.float32),
        ],
    )
    def w_kernel(sd_hbm, atbl_hbm, w_hbm, den_hbm,
                 sd_i, asb, adb, wb, sem_g, den_sp):
        c = lax.axis_index("c")
        s = lax.axis_index("s")
        wid = s * 2 + c
        lane = lax.iota(jnp.int32, 16)
        zf = jnp.zeros((16,), jnp.float32)

        @pl.loop(0, 128)
        def _(t):
            r = t * 2 + lax.shift_right_logical(lane, 3)
            plsc.store_scatter(wb, [r, lane & 7], zf)

        @pl.loop(0, 25)
        def _(t):
            g = t * 16 + s

            @pl.when(g < NROWBLK)
            def _():
                pltpu.sync_copy(wb.at[pl.ds(0, 128)],
                                den_sp.at[pl.ds(g * 128, 128)])

        plsc.subcore_barrier()

        ebase = wid * (EPT_A // 128)
        wbase = wid * EPT_A

        def issue(jj, slot):
            r0 = ebase + jj * 2
            pltpu.sync_copy(sd_hbm.at[pl.ds(r0, 2)], sd_i.at[slot])
            for q in range(2):
                pltpu.async_copy(
                    atbl_hbm.at[sd_i.at[slot, q, 0]], asb.at[slot, q], sem_g)
                pltpu.async_copy(
                    atbl_hbm.at[sd_i.at[slot, q, 1]], adb.at[slot, q], sem_g)

        def drain(slot):
            for q in range(2):
                pltpu.make_async_copy(
                    atbl_hbm.at[pl.ds(0, 128)], asb.at[slot, q], sem_g).wait()
                pltpu.make_async_copy(
                    atbl_hbm.at[pl.ds(0, 128)], adb.at[slot, q], sem_g).wait()

        issue(0, 0)

        @pl.loop(0, NCHUNK_A, step=2)
        def _(j):
            for slot in range(2):
                jj = j + slot
                issue(jnp.minimum(jj + 1, NCHUNK_A - 1), 1 - slot)
                drain(slot)
                for q in range(2):
                    iq = jnp.full((16,), q, jnp.int32)
                    if l1_mode:
                        @pl.loop(0, 64)
                        def _(t):
                            r = t * 2 + lax.shift_right_logical(lane, 3)
                            hh = lane & 7
                            a = (plsc.load_gather(asb.at[slot], [iq, r, hh])
                                 + plsc.load_gather(adb.at[slot],
                                                    [iq, r, 8 + hh]))
                            w = jnp.exp(jnp.maximum(a, 0.2 * a))
                            plsc.store_scatter(wb, [q * 128 + r, hh], w)
                    else:
                        zi = jnp.zeros((16,), jnp.int32)

                        @pl.loop(0, 8)
                        def _(t):
                            r = t * 16 + lane
                            a = (plsc.load_gather(asb.at[slot], [iq, r, zi])
                                 + plsc.load_gather(adb.at[slot],
                                                    [iq, r, zi + 1]))
                            w = jnp.exp(jnp.maximum(a, 0.2 * a))
                            plsc.store_scatter(wb, [q * 128 + r, zi], w)

                for q in range(2):
                    pltpu.sync_copy(wb.at[pl.ds(q * 128, 128)],
                                    den_sp.at[sd_i.at[slot, q, 1]], add=True)
                pltpu.sync_copy(
                    wb, w_hbm.at[pl.ds(wbase + jj * K, K), :])

        drain(0)
        plsc.subcore_barrier()

        @pl.loop(0, 25)
        def _(t):
            g = t * 16 + s

            @pl.when(g < NROWBLK)
            def _():
                r = g * 128
                pltpu.sync_copy(den_sp.at[pl.ds(r, 128)],
                                den_hbm.at[c, pl.ds(r, 128), :])

    return w_kernel


def _make_msg_kernel(l1_mode):
    @functools.partial(
        pl.kernel,
        out_type=jax.ShapeDtypeStruct((2, NPAD, 32), jnp.float32),
        mesh=_mesh(),
        compiler_params=_SC_PARAMS,
        scratch_types=[
            pltpu.VMEM((2, 2, 2, 128), jnp.int32),
            pltpu.VMEM((2, 256, 8), jnp.float32),
            pltpu.VMEM((2, 2, 128, 32), jnp.float32),
            pltpu.VMEM((2, 128, 32), jnp.float32),
            pltpu.SemaphoreType.DMA,
            pltpu.SemaphoreType.DMA,
            pltpu.VMEM_SHARED((NPAD, 32), jnp.float32),
        ],
    )
    def msg_kernel(sd_hbm, w_hbm, flo_hbm, fhi_hbm, out_hbm,
                   sd_i, wb, fb, msgb, sem_g, sem_w, out_sp):
        c = lax.axis_index("c")
        s = lax.axis_index("s")
        lane = lax.iota(jnp.int32, 16)
        zi = jnp.zeros((16,), jnp.int32)
        zf = jnp.zeros((16,), jnp.float32)

        for q in range(2):
            iq = jnp.full((16,), q, jnp.int32)

            @pl.loop(0, 128)
            def _(e):
                ie = zi + e
                plsc.store_scatter(msgb, [iq, ie, lane], zf)
                plsc.store_scatter(msgb, [iq, ie, lane + 16], zf)

        @pl.loop(0, 25)
        def _(t):
            g = t * 16 + s

            @pl.when(g < NROWBLK)
            def _():
                pltpu.sync_copy(msgb.at[0], out_sp.at[pl.ds(g * 128, 128)])

        plsc.subcore_barrier()

        ebase = s * (EPT_B // 128)
        wbase = s * EPT_B

        def issue(jj, slot):
            r0 = ebase + jj * 2
            pltpu.sync_copy(sd_hbm.at[pl.ds(r0, 2)], sd_i.at[slot])
            pltpu.async_copy(
                w_hbm.at[pl.ds(wbase + jj * K, K), :], wb.at[slot], sem_w)
            for q in range(2):
                @pl.when(c == 0)
                def _():
                    pltpu.async_copy(
                        flo_hbm.at[sd_i.at[slot, q, 0]], fb.at[slot, q], sem_g)

                @pl.when(c == 1)
                def _():
                    pltpu.async_copy(
                        fhi_hbm.at[sd_i.at[slot, q, 0]], fb.at[slot, q], sem_g)

        def drain(slot):
            pltpu.make_async_copy(
                w_hbm.at[pl.ds(0, K), :], wb.at[slot], sem_w).wait()
            for q in range(2):
                pltpu.make_async_copy(
                    flo_hbm.at[pl.ds(0, 128)], fb.at[slot, q], sem_g).wait()

        issue(0, 0)

        if l1_mode:
            wcol0 = c * 4 + lax.shift_right_logical(lane, 3)
            wcol1 = wcol0 + 2
        else:
            wcol0 = zi
            wcol1 = zi

        @pl.loop(0, NCHUNK_B, step=2)
        def _(j):
            for slot in range(2):
                jj = j + slot
                issue(jnp.minimum(jj + 1, NCHUNK_B - 1), 1 - slot)
                drain(slot)

                for q in range(2):
                    @pl.loop(0, 128)
                    def _(e):
                        iw = zi + (e + q * 128)
                        w0 = plsc.load_gather(wb.at[slot], [iw, wcol0])
                        f0 = fb[slot, q, e, pl.ds(0, 16)]
                        msgb[q, e, pl.ds(0, 16)] = w0 * f0
                        w1 = plsc.load_gather(wb.at[slot], [iw, wcol1])
                        f1 = fb[slot, q, e, pl.ds(16, 16)]
                        msgb[q, e, pl.ds(16, 16)] = w1 * f1

                for q in range(2):
                    pltpu.sync_copy(msgb.at[q],
                                    out_sp.at[sd_i.at[slot, q, 1]], add=True)

        drain(0)
        plsc.subcore_barrier()

        @pl.loop(0, 25)
        def _(t):
            g = t * 16 + s

            @pl.when(g < NROWBLK)
            def _():
                r = g * 128
                pltpu.sync_copy(out_sp.at[pl.ds(r, 128)],
                                out_hbm.at[c, pl.ds(r, 128), :])

    return msg_kernel


_w_l1 = _make_w_kernel(True)
_w_single = _make_w_kernel(False)
_msg_l1 = _make_msg_kernel(True)
_msg_single = _make_msg_kernel(False)


def _tc1_body(x_ref, w1_ref, as_ref, ad_ref, hlo_ref, hhi_ref, atbl_ref):
    h = jnp.dot(x_ref[...], w1_ref[...], preferred_element_type=jnp.float32)
    a_s = jnp.dot(h, as_ref[...], preferred_element_type=jnp.float32)
    a_d = jnp.dot(h, ad_ref[...], preferred_element_type=jnp.float32)
    hlo_ref[...] = h[:, 0:32]
    hhi_ref[...] = h[:, 32:64]
    atbl_ref[...] = jnp.concatenate([a_s, a_d], axis=1)


def _tc2_body(p0_ref, p1_ref, d0_ref, d1_ref, rep_ref, b_ref, w_ref, m_ref,
              flo_ref, fhi_ref, atbl_ref):
    rep = rep_ref[...]
    den = d0_ref[...] + d1_ref[...] + 1e-16
    dd0 = jnp.dot(den[:, 0:4], rep, preferred_element_type=jnp.float32)
    dd1 = jnp.dot(den[:, 4:8], rep, preferred_element_type=jnp.float32)
    x = jnp.concatenate([p0_ref[...] / dd0, p1_ref[...] / dd1],
                        axis=1) + b_ref[...]
    x = _elu(x)
    h = jnp.dot(x, w_ref[...], preferred_element_type=jnp.float32)
    flo_ref[...] = h[:, 0:32]
    fhi_ref[...] = h[:, 32:64]
    atbl_ref[...] = jnp.dot(x, m_ref[...], preferred_element_type=jnp.float32)


def _tc3_body(p0_ref, p1_ref, d0_ref, d1_ref, b_ref, m_ref, flo_ref, fhi_ref,
              atbl_ref):
    d = d0_ref[...][:, 0:1] + d1_ref[...][:, 0:1] + 1e-16
    x = jnp.concatenate([p0_ref[...] / d, p1_ref[...] / d], axis=1) + b_ref[...]
    x = _elu(x)
    flo_ref[...] = x[:, 0:32]
    fhi_ref[...] = x[:, 32:64]
    atbl_ref[...] = jnp.dot(x, m_ref[...], preferred_element_type=jnp.float32)


def _tc4_body(p0_ref, p1_ref, d0_ref, d1_ref, w_ref, b_ref, out_ref):
    d = d0_ref[...][:, 0:1] + d1_ref[...][:, 0:1] + 1e-16
    agg = jnp.concatenate([p0_ref[...] / d, p1_ref[...] / d], axis=1)
    out_ref[...] = jnp.dot(agg, w_ref[...],
                           preferred_element_type=jnp.float32) + b_ref[...]


def _row_spec(c):
    return pl.BlockSpec((BLK, c), lambda i: (i, 0))


def _full_spec(r, c):
    return pl.BlockSpec((r, c), lambda i: (0, 0))


_tc1 = pl.pallas_call(
    _tc1_body,
    grid=(GRID,),
    in_specs=[_row_spec(128), _full_spec(128, 64), _full_spec(64, 8),
              _full_spec(64, 8)],
    out_specs=[_row_spec(32), _row_spec(32), _row_spec(16)],
    out_shape=[jax.ShapeDtypeStruct((NPAD, 32), jnp.float32),
               jax.ShapeDtypeStruct((NPAD, 32), jnp.float32),
               jax.ShapeDtypeStruct((NPAD, 16), jnp.float32)],
)

_tc2 = pl.pallas_call(
    _tc2_body,
    grid=(GRID,),
    in_specs=[_row_spec(32), _row_spec(32), _row_spec(8), _row_spec(8),
              _full_spec(4, 32), _full_spec(1, 64), _full_spec(64, 64),
              _full_spec(64, 16)],
    out_specs=[_row_spec(32), _row_spec(32), _row_spec(16)],
    out_shape=[jax.ShapeDtypeStruct((NPAD, 32), jnp.float32),
               jax.ShapeDtypeStruct((NPAD, 32), jnp.float32),
               jax.ShapeDtypeStruct((NPAD, 16), jnp.float32)],
)

_tc3 = pl.pallas_call(
    _tc3_body,
    grid=(GRID,),
    in_specs=[_row_spec(32), _row_spec(32), _row_spec(8), _row_spec(8),
              _full_spec(1, 64), _full_spec(64, 16)],
    out_specs=[_row_spec(32), _row_spec(32), _row_spec(16)],
    out_shape=[jax.ShapeDtypeStruct((NPAD, 32), jnp.float32),
               jax.ShapeDtypeStruct((NPAD, 32), jnp.float32),
               jax.ShapeDtypeStruct((NPAD, 16), jnp.float32)],
)

_tc4 = pl.pallas_call(
    _tc4_body,
    grid=(GRID,),
    in_specs=[_row_spec(32), _row_spec(32), _row_spec(8), _row_spec(8),
              _full_spec(64, 128), _full_spec(1, 128)],
    out_specs=_row_spec(128),
    out_shape=jax.ShapeDtypeStruct((NPAD, 128), jnp.float32),
)


def kernel(x, edge_index, W1, a_src1, a_dst1, b1, W2, a_src2, a_dst2, b2,
           W3, a_src3, a_dst3, b3):
    f32 = jnp.float32
    xp = jnp.pad(x.astype(f32), ((0, NPAD - N_NODES), (0, 128 - x.shape[1])))
    w1p = jnp.pad(W1, ((0, 128 - W1.shape[0]), (0, 0)))
    eye8 = jnp.eye(8, dtype=f32)
    as1 = (a_src1[0][:, :, None] * eye8[:, None, :]).reshape(64, 8)
    ad1 = (a_dst1[0][:, :, None] * eye8[:, None, :]).reshape(64, 8)
    m2 = jnp.zeros((64, 16), f32)
    m2 = m2.at[:, 0].set(W2 @ a_src2[0, 0]).at[:, 1].set(W2 @ a_dst2[0, 0])
    m3 = jnp.zeros((64, 16), f32)
    m3 = m3.at[:, 0].set(W3 @ a_src3[0, 0]).at[:, 1].set(W3 @ a_dst3[0, 0])
    rep = jnp.repeat(jnp.eye(4, dtype=f32), 8, axis=1)
    w3p = jnp.pad(W3, ((0, 0), (0, 128 - W3.shape[1])))
    b3p = jnp.pad(b3, (0, 128 - b3.shape[0])).reshape(1, 128)
    b1r = b1.reshape(1, 64)
    b2r = b2.reshape(1, 64)
    src = jnp.concatenate(
        [edge_index[0], jnp.zeros((EPAD - N_EDGES,), jnp.int32)])
    dst = jnp.concatenate(
        [edge_index[1], jnp.full((EPAD - N_EDGES,), DUMP_ROW, jnp.int32)])
    sd2d = jnp.stack(
        [src.reshape(EPAD // 128, 128), dst.reshape(EPAD // 128, 128)],
        axis=1)

    h1lo, h1hi, atbl1 = _tc1(xp, w1p, as1, ad1)
    w1e, den1 = _w_l1(sd2d, atbl1)
    out1 = _msg_l1(sd2d, w1e, h1lo, h1hi)
    f2lo, f2hi, atbl2 = _tc2(out1[0], out1[1], den1[0], den1[1], rep, b1r,
                             W2, m2)
    w2e, den2 = _w_single(sd2d, atbl2)
    out2 = _msg_single(sd2d, w2e, f2lo, f2hi)
    f3lo, f3hi, atbl3 = _tc3(out2[0], out2[1], den2[0], den2[1], b2r, m3)
    w3e, den3 = _w_single(sd2d, atbl3)
    out3 = _msg_single(sd2d, w3e, f3lo, f3hi)
    res = _tc4(out3[0], out3[1], den3[0], den3[1], w3p, b3p)
    return res[:N_NODES, :121]


# speedup vs baseline: 44.8313x; 1.0137x over previous
import functools

import jax
import jax.numpy as jnp
from jax import lax
from jax.experimental import pallas as pl
from jax.experimental.pallas import tpu as pltpu
from jax.experimental.pallas import tpu_sc as plsc

N_NODES = 50000
N_EDGES = 800000
NPAD = 50176
NROWBLK = NPAD // 128
K = 256
NCHUNK_A = 98
NCHUNK_B = 196
EPT_A = NCHUNK_A * K
EPT_B = NCHUNK_B * K
EPAD = EPT_B * 16
DUMP_ROW = N_NODES
BLK = 256
GRID = NPAD // BLK

_SC_PARAMS = pltpu.CompilerParams(
    needs_layout_passes=False, use_tc_tiling_on_sc=False)


def _elu(v):
    return jnp.where(v > 0, v, jnp.exp(jnp.minimum(v, 0.0)) - 1.0)


def _mesh():
    return plsc.VectorSubcoreMesh(core_axis_name="c", subcore_axis_name="s")


def _make_w_kernel(l1_mode):
    @functools.partial(
        pl.kernel,
        out_type=(jax.ShapeDtypeStruct((EPAD + K, 8), jnp.float32),
                  jax.ShapeDtypeStruct((2, NPAD, 8), jnp.float32)),
        mesh=_mesh(),
        compiler_params=_SC_PARAMS,
        scratch_types=[
            pltpu.VMEM((2, 2, 2, 128), jnp.int32),
            pltpu.VMEM((2, 2, 128, 16), jnp.float32),
            pltpu.VMEM((2, 2, 128, 16), jnp.float32),
            pltpu.VMEM((256, 8), jnp.float32),
            pltpu.SemaphoreType.DMA,
            pltpu.VMEM_SHARED((NPAD, 8), jnp.float32),
        ],
    )
    def w_kernel(sd_hbm, atbl_hbm, w_hbm, den_hbm,
                 sd_i, asb, adb, wb, sem_g, den_sp):
        c = lax.axis_index("c")
        s = lax.axis_index("s")
        wid = s * 2 + c
        lane = lax.iota(jnp.int32, 16)
        zf = jnp.zeros((16,), jnp.float32)

        @pl.loop(0, 128)
        def _(t):
            r = t * 2 + lax.shift_right_logical(lane, 3)
            plsc.store_scatter(wb, [r, lane & 7], zf)

        @pl.loop(0, 25)
        def _(t):
            g = t * 16 + s

            @pl.when(g < NROWBLK)
            def _():
                pltpu.sync_copy(wb.at[pl.ds(0, 128)],
                                den_sp.at[pl.ds(g * 128, 128)])

        plsc.subcore_barrier()

        ebase = wid * (EPT_A // 128)
        wbase = wid * EPT_A

        def issue(jj, slot):
            r0 = ebase + jj * 2
            pltpu.sync_copy(sd_hbm.at[pl.ds(r0, 2)], sd_i.at[slot])
            for q in range(2):
                pltpu.async_copy(
                    atbl_hbm.at[sd_i.at[slot, q, 0]], asb.at[slot, q], sem_g)
                pltpu.async_copy(
                    atbl_hbm.at[sd_i.at[slot, q, 1]], adb.at[slot, q], sem_g)

        def drain(slot):
            for q in range(2):
                pltpu.make_async_copy(
                    atbl_hbm.at[pl.ds(0, 128)], asb.at[slot, q], sem_g).wait()
                pltpu.make_async_copy(
                    atbl_hbm.at[pl.ds(0, 128)], adb.at[slot, q], sem_g).wait()

        issue(0, 0)

        @pl.loop(0, NCHUNK_A, step=2)
        def _(j):
            for slot in range(2):
                jj = j + slot
                issue(jnp.minimum(jj + 1, NCHUNK_A - 1), 1 - slot)
                drain(slot)
                for q in range(2):
                    iq = jnp.full((16,), q, jnp.int32)
                    if l1_mode:
                        @pl.loop(0, 64)
                        def _(t):
                            r = t * 2 + lax.shift_right_logical(lane, 3)
                            hh = lane & 7
                            a = (plsc.load_gather(asb.at[slot], [iq, r, hh])
                                 + plsc.load_gather(adb.at[slot],
                                                    [iq, r, 8 + hh]))
                            w = jnp.exp(jnp.maximum(a, 0.2 * a))
                            plsc.store_scatter(wb, [q * 128 + r, hh], w)
                    else:
                        zi = jnp.zeros((16,), jnp.int32)

                        @pl.loop(0, 8)
                        def _(t):
                            r = t * 16 + lane
                            a = (plsc.load_gather(asb.at[slot], [iq, r, zi])
                                 + plsc.load_gather(adb.at[slot],
                                                    [iq, r, zi + 1]))
                            w = jnp.exp(jnp.maximum(a, 0.2 * a))
                            plsc.store_scatter(wb, [q * 128 + r, zi], w)

                for q in range(2):
                    pltpu.sync_copy(wb.at[pl.ds(q * 128, 128)],
                                    den_sp.at[sd_i.at[slot, q, 1]], add=True)
                pltpu.sync_copy(
                    wb, w_hbm.at[pl.ds(wbase + jj * K, K), :])

        drain(0)
        plsc.subcore_barrier()

        @pl.loop(0, 25)
        def _(t):
            g = t * 16 + s

            @pl.when(g < NROWBLK)
            def _():
                r = g * 128
                pltpu.sync_copy(den_sp.at[pl.ds(r, 128)],
                                den_hbm.at[c, pl.ds(r, 128), :])

    return w_kernel


def _make_msg_kernel(l1_mode):
    @functools.partial(
        pl.kernel,
        out_type=jax.ShapeDtypeStruct((2, NPAD, 32), jnp.float32),
        mesh=_mesh(),
        compiler_params=_SC_PARAMS,
        scratch_types=[
            pltpu.VMEM((2, 2, 2, 128), jnp.int32),
            pltpu.VMEM((2, 256, 8), jnp.float32),
            pltpu.VMEM((2, 2, 128, 32), jnp.float32),
            pltpu.VMEM((2, 128, 32), jnp.float32),
            pltpu.SemaphoreType.DMA,
            pltpu.SemaphoreType.DMA,
            pltpu.VMEM_SHARED((NPAD, 32), jnp.float32),
        ],
    )
    def msg_kernel(sd_hbm, w_hbm, flo_hbm, fhi_hbm, out_hbm,
                   sd_i, wb, fb, msgb, sem_g, sem_w, out_sp):
        c = lax.axis_index("c")
        s = lax.axis_index("s")
        lane = lax.iota(jnp.int32, 16)
        zi = jnp.zeros((16,), jnp.int32)
        zf = jnp.zeros((16,), jnp.float32)

        for q in range(2):
            iq = jnp.full((16,), q, jnp.int32)

            @pl.loop(0, 128)
            def _(e):
                ie = zi + e
                plsc.store_scatter(msgb, [iq, ie, lane], zf)
                plsc.store_scatter(msgb, [iq, ie, lane + 16], zf)

        @pl.loop(0, 25)
        def _(t):
            g = t * 16 + s

            @pl.when(g < NROWBLK)
            def _():
                pltpu.sync_copy(msgb.at[0], out_sp.at[pl.ds(g * 128, 128)])

        plsc.subcore_barrier()

        ebase = s * (EPT_B // 128)
        wbase = s * EPT_B

        def issue(jj, slot):
            r0 = ebase + jj * 2
            pltpu.sync_copy(sd_hbm.at[pl.ds(r0, 2)], sd_i.at[slot])
            pltpu.async_copy(
                w_hbm.at[pl.ds(wbase + jj * K, K), :], wb.at[slot], sem_w)
            for q in range(2):
                @pl.when(c == 0)
                def _():
                    pltpu.async_copy(
                        flo_hbm.at[sd_i.at[slot, q, 0]], fb.at[slot, q], sem_g)

                @pl.when(c == 1)
                def _():
                    pltpu.async_copy(
                        fhi_hbm.at[sd_i.at[slot, q, 0]], fb.at[slot, q], sem_g)

        def drain(slot):
            pltpu.make_async_copy(
                w_hbm.at[pl.ds(0, K), :], wb.at[slot], sem_w).wait()
            for q in range(2):
                pltpu.make_async_copy(
                    flo_hbm.at[pl.ds(0, 128)], fb.at[slot, q], sem_g).wait()

        issue(0, 0)

        if l1_mode:
            wcol0 = c * 4 + lax.shift_right_logical(lane, 3)
            wcol1 = wcol0 + 2
        else:
            wcol0 = zi
            wcol1 = zi

        @pl.loop(0, NCHUNK_B, step=2)
        def _(j):
            for slot in range(2):
                jj = j + slot
                issue(jnp.minimum(jj + 1, NCHUNK_B - 1), 1 - slot)
                drain(slot)

                for q in range(2):
                    @pl.loop(0, 128)
                    def _(e):
                        iw = zi + (e + q * 128)
                        w0 = plsc.load_gather(wb.at[slot], [iw, wcol0])
                        f0 = fb[slot, q, e, pl.ds(0, 16)]
                        msgb[q, e, pl.ds(0, 16)] = w0 * f0
                        if l1_mode:
                            w1 = plsc.load_gather(wb.at[slot], [iw, wcol1])
                        else:
                            w1 = w0
                        f1 = fb[slot, q, e, pl.ds(16, 16)]
                        msgb[q, e, pl.ds(16, 16)] = w1 * f1

                for q in range(2):
                    pltpu.sync_copy(msgb.at[q],
                                    out_sp.at[sd_i.at[slot, q, 1]], add=True)

        drain(0)
        plsc.subcore_barrier()

        @pl.loop(0, 25)
        def _(t):
            g = t * 16 + s

            @pl.when(g < NROWBLK)
            def _():
                r = g * 128
                pltpu.sync_copy(out_sp.at[pl.ds(r, 128)],
                                out_hbm.at[c, pl.ds(r, 128), :])

    return msg_kernel


_w_l1 = _make_w_kernel(True)
_w_single = _make_w_kernel(False)
_msg_l1 = _make_msg_kernel(True)
_msg_single = _make_msg_kernel(False)


def _tc1_body(x_ref, w1_ref, as_ref, ad_ref, hlo_ref, hhi_ref, atbl_ref):
    h = jnp.dot(x_ref[...], w1_ref[...], preferred_element_type=jnp.float32)
    a_s = jnp.dot(h, as_ref[...], preferred_element_type=jnp.float32)
    a_d = jnp.dot(h, ad_ref[...], preferred_element_type=jnp.float32)
    hlo_ref[...] = h[:, 0:32]
    hhi_ref[...] = h[:, 32:64]
    atbl_ref[...] = jnp.concatenate([a_s, a_d], axis=1)


def _tc2_body(p0_ref, p1_ref, d0_ref, d1_ref, rep_ref, b_ref, w_ref, m_ref,
              flo_ref, fhi_ref, atbl_ref):
    rep = rep_ref[...]
    den = d0_ref[...] + d1_ref[...] + 1e-16
    dd0 = jnp.dot(den[:, 0:4], rep, preferred_element_type=jnp.float32)
    dd1 = jnp.dot(den[:, 4:8], rep, preferred_element_type=jnp.float32)
    x = jnp.concatenate([p0_ref[...] / dd0, p1_ref[...] / dd1],
                        axis=1) + b_ref[...]
    x = _elu(x)
    h = jnp.dot(x, w_ref[...], preferred_element_type=jnp.float32)
    flo_ref[...] = h[:, 0:32]
    fhi_ref[...] = h[:, 32:64]
    atbl_ref[...] = jnp.dot(x, m_ref[...], preferred_element_type=jnp.float32)


def _tc3_body(p0_ref, p1_ref, d0_ref, d1_ref, b_ref, m_ref, flo_ref, fhi_ref,
              atbl_ref):
    d = d0_ref[...][:, 0:1] + d1_ref[...][:, 0:1] + 1e-16
    x = jnp.concatenate([p0_ref[...] / d, p1_ref[...] / d], axis=1) + b_ref[...]
    x = _elu(x)
    flo_ref[...] = x[:, 0:32]
    fhi_ref[...] = x[:, 32:64]
    atbl_ref[...] = jnp.dot(x, m_ref[...], preferred_element_type=jnp.float32)


def _tc4_body(p0_ref, p1_ref, d0_ref, d1_ref, w_ref, b_ref, out_ref):
    d = d0_ref[...][:, 0:1] + d1_ref[...][:, 0:1] + 1e-16
    agg = jnp.concatenate([p0_ref[...] / d, p1_ref[...] / d], axis=1)
    out_ref[...] = jnp.dot(agg, w_ref[...],
                           preferred_element_type=jnp.float32) + b_ref[...]


def _row_spec(c):
    return pl.BlockSpec((BLK, c), lambda i: (i, 0))


def _full_spec(r, c):
    return pl.BlockSpec((r, c), lambda i: (0, 0))


_tc1 = pl.pallas_call(
    _tc1_body,
    grid=(GRID,),
    in_specs=[_row_spec(128), _full_spec(128, 64), _full_spec(64, 8),
              _full_spec(64, 8)],
    out_specs=[_row_spec(32), _row_spec(32), _row_spec(16)],
    out_shape=[jax.ShapeDtypeStruct((NPAD, 32), jnp.float32),
               jax.ShapeDtypeStruct((NPAD, 32), jnp.float32),
               jax.ShapeDtypeStruct((NPAD, 16), jnp.float32)],
)

_tc2 = pl.pallas_call(
    _tc2_body,
    grid=(GRID,),
    in_specs=[_row_spec(32), _row_spec(32), _row_spec(8), _row_spec(8),
              _full_spec(4, 32), _full_spec(1, 64), _full_spec(64, 64),
              _full_spec(64, 16)],
    out_specs=[_row_spec(32), _row_spec(32), _row_spec(16)],
    out_shape=[jax.ShapeDtypeStruct((NPAD, 32), jnp.float32),
               jax.ShapeDtypeStruct((NPAD, 32), jnp.float32),
               jax.ShapeDtypeStruct((NPAD, 16), jnp.float32)],
)

_tc3 = pl.pallas_call(
    _tc3_body,
    grid=(GRID,),
    in_specs=[_row_spec(32), _row_spec(32), _row_spec(8), _row_spec(8),
              _full_spec(1, 64), _full_spec(64, 16)],
    out_specs=[_row_spec(32), _row_spec(32), _row_spec(16)],
    out_shape=[jax.ShapeDtypeStruct((NPAD, 32), jnp.float32),
               jax.ShapeDtypeStruct((NPAD, 32), jnp.float32),
               jax.ShapeDtypeStruct((NPAD, 16), jnp.float32)],
)

_tc4 = pl.pallas_call(
    _tc4_body,
    grid=(GRID,),
    in_specs=[_row_spec(32), _row_spec(32), _row_spec(8), _row_spec(8),
              _full_spec(64, 128), _full_spec(1, 128)],
    out_specs=_row_spec(128),
    out_shape=jax.ShapeDtypeStruct((NPAD, 128), jnp.float32),
)


def kernel(x, edge_index, W1, a_src1, a_dst1, b1, W2, a_src2, a_dst2, b2,
           W3, a_src3, a_dst3, b3):
    f32 = jnp.float32
    xp = jnp.pad(x.astype(f32), ((0, NPAD - N_NODES), (0, 128 - x.shape[1])))
    w1p = jnp.pad(W1, ((0, 128 - W1.shape[0]), (0, 0)))
    eye8 = jnp.eye(8, dtype=f32)
    as1 = (a_src1[0][:, :, None] * eye8[:, None, :]).reshape(64, 8)
    ad1 = (a_dst1[0][:, :, None] * eye8[:, None, :]).reshape(64, 8)
    m2 = jnp.zeros((64, 16), f32)
    m2 = m2.at[:, 0].set(W2 @ a_src2[0, 0]).at[:, 1].set(W2 @ a_dst2[0, 0])
    m3 = jnp.zeros((64, 16), f32)
    m3 = m3.at[:, 0].set(W3 @ a_src3[0, 0]).at[:, 1].set(W3 @ a_dst3[0, 0])
    rep = jnp.repeat(jnp.eye(4, dtype=f32), 8, axis=1)
    w3p = jnp.pad(W3, ((0, 0), (0, 128 - W3.shape[1])))
    b3p = jnp.pad(b3, (0, 128 - b3.shape[0])).reshape(1, 128)
    b1r = b1.reshape(1, 64)
    b2r = b2.reshape(1, 64)
    src = jnp.concatenate(
        [edge_index[0], jnp.zeros((EPAD - N_EDGES,), jnp.int32)])
    dst = jnp.concatenate(
        [edge_index[1], jnp.full((EPAD - N_EDGES,), DUMP_ROW, jnp.int32)])
    sd2d = jnp.stack(
        [src.reshape(EPAD // 128, 128), dst.reshape(EPAD // 128, 128)],
        axis=1)

    h1lo, h1hi, atbl1 = _tc1(xp, w1p, as1, ad1)
    w1e, den1 = _w_l1(sd2d, atbl1)
    out1 = _msg_l1(sd2d, w1e, h1lo, h1hi)
    f2lo, f2hi, atbl2 = _tc2(out1[0], out1[1], den1[0], den1[1], rep, b1r,
                             W2, m2)
    w2e, den2 = _w_single(sd2d, atbl2)
    out2 = _msg_single(sd2d, w2e, f2lo, f2hi)
    f3lo, f3hi, atbl3 = _tc3(out2[0], out2[1], den2[0], den2[1], b2r, m3)
    w3e, den3 = _w_single(sd2d, atbl3)
    out3 = _msg_single(sd2d, w3e, f3lo, f3hi)
    res = _tc4(out3[0], out3[1], den3[0], den3[1], w3p, b3p)
    return res[:N_NODES, :121]


# speedup vs baseline: 45.3212x; 1.0109x over previous
import functools

import jax
import jax.numpy as jnp
from jax import lax
from jax.experimental import pallas as pl
from jax.experimental.pallas import tpu as pltpu
from jax.experimental.pallas import tpu_sc as plsc

N_NODES = 50000
N_EDGES = 800000
NPAD = 50176
NROWBLK = NPAD // 128
K = 256
NCHUNK_A = 98
NCHUNK_B = 196
EPT_A = NCHUNK_A * K
EPT_B = NCHUNK_B * K
EPAD = EPT_B * 16
DUMP_ROW = N_NODES
BLK = 256
GRID = NPAD // BLK

_SC_PARAMS = pltpu.CompilerParams(
    needs_layout_passes=False, use_tc_tiling_on_sc=False)


def _elu(v):
    return jnp.where(v > 0, v, jnp.exp(jnp.minimum(v, 0.0)) - 1.0)


def _mesh():
    return plsc.VectorSubcoreMesh(core_axis_name="c", subcore_axis_name="s")


def _make_w_kernel(l1_mode):
    @functools.partial(
        pl.kernel,
        out_type=(jax.ShapeDtypeStruct((EPAD + K, 8), jnp.float32),
                  jax.ShapeDtypeStruct((2, NPAD, 8), jnp.float32)),
        mesh=_mesh(),
        compiler_params=_SC_PARAMS,
        scratch_types=[
            pltpu.VMEM((2, 2, 2, 128), jnp.int32),
            pltpu.VMEM((2, 2, 128, 16), jnp.float32),
            pltpu.VMEM((2, 2, 128, 16), jnp.float32),
            pltpu.VMEM((256, 8), jnp.float32),
            pltpu.SemaphoreType.DMA,
            pltpu.VMEM_SHARED((NPAD, 8), jnp.float32),
        ],
    )
    def w_kernel(sd_hbm, atbl_hbm, w_hbm, den_hbm,
                 sd_i, asb, adb, wb, sem_g, den_sp):
        c = lax.axis_index("c")
        s = lax.axis_index("s")
        wid = s * 2 + c
        lane = lax.iota(jnp.int32, 16)
        zf = jnp.zeros((16,), jnp.float32)

        @pl.loop(0, 128)
        def _(t):
            r = t * 2 + lax.shift_right_logical(lane, 3)
            plsc.store_scatter(wb, [r, lane & 7], zf)

        @pl.loop(0, 25)
        def _(t):
            g = t * 16 + s

            @pl.when(g < NROWBLK)
            def _():
                pltpu.sync_copy(wb.at[pl.ds(0, 128)],
                                den_sp.at[pl.ds(g * 128, 128)])

        plsc.subcore_barrier()

        ebase = wid * (EPT_A // 128)
        wbase = wid * EPT_A

        def issue(jj, slot):
            r0 = ebase + jj * 2
            pltpu.sync_copy(sd_hbm.at[pl.ds(r0, 2)], sd_i.at[slot])
            for q in range(2):
                pltpu.async_copy(
                    atbl_hbm.at[sd_i.at[slot, q, 0]], asb.at[slot, q], sem_g)
                pltpu.async_copy(
                    atbl_hbm.at[sd_i.at[slot, q, 1]], adb.at[slot, q], sem_g)

        def drain(slot):
            for q in range(2):
                pltpu.make_async_copy(
                    atbl_hbm.at[pl.ds(0, 128)], asb.at[slot, q], sem_g).wait()
                pltpu.make_async_copy(
                    atbl_hbm.at[pl.ds(0, 128)], adb.at[slot, q], sem_g).wait()

        issue(0, 0)

        @pl.loop(0, NCHUNK_A, step=2)
        def _(j):
            for slot in range(2):
                jj = j + slot
                issue(jnp.minimum(jj + 1, NCHUNK_A - 1), 1 - slot)
                drain(slot)
                for q in range(2):
                    iq = jnp.full((16,), q, jnp.int32)
                    if l1_mode:
                        @pl.loop(0, 64, unroll=4)
                        def _(t):
                            r = t * 2 + lax.shift_right_logical(lane, 3)
                            hh = lane & 7
                            a = (plsc.load_gather(asb.at[slot], [iq, r, hh])
                                 + plsc.load_gather(adb.at[slot],
                                                    [iq, r, 8 + hh]))
                            w = jnp.exp(jnp.maximum(a, 0.2 * a))
                            plsc.store_scatter(wb, [q * 128 + r, hh], w)
                    else:
                        zi = jnp.zeros((16,), jnp.int32)

                        @pl.loop(0, 8)
                        def _(t):
                            r = t * 16 + lane
                            a = (plsc.load_gather(asb.at[slot], [iq, r, zi])
                                 + plsc.load_gather(adb.at[slot],
                                                    [iq, r, zi + 1]))
                            w = jnp.exp(jnp.maximum(a, 0.2 * a))
                            plsc.store_scatter(wb, [q * 128 + r, zi], w)

                for q in range(2):
                    pltpu.sync_copy(wb.at[pl.ds(q * 128, 128)],
                                    den_sp.at[sd_i.at[slot, q, 1]], add=True)
                pltpu.sync_copy(
                    wb, w_hbm.at[pl.ds(wbase + jj * K, K), :])

        drain(0)
        plsc.subcore_barrier()

        @pl.loop(0, 25)
        def _(t):
            g = t * 16 + s

            @pl.when(g < NROWBLK)
            def _():
                r = g * 128
                pltpu.sync_copy(den_sp.at[pl.ds(r, 128)],
                                den_hbm.at[c, pl.ds(r, 128), :])

    return w_kernel


def _make_msg_kernel(l1_mode):
    @functools.partial(
        pl.kernel,
        out_type=jax.ShapeDtypeStruct((2, NPAD, 32), jnp.float32),
        mesh=_mesh(),
        compiler_params=_SC_PARAMS,
        scratch_types=[
            pltpu.VMEM((2, 2, 2, 128), jnp.int32),
            pltpu.VMEM((2, 256, 8), jnp.float32),
            pltpu.VMEM((2, 2, 128, 32), jnp.float32),
            pltpu.VMEM((2, 128, 32), jnp.float32),
            pltpu.SemaphoreType.DMA,
            pltpu.SemaphoreType.DMA,
            pltpu.VMEM_SHARED((NPAD, 32), jnp.float32),
        ],
    )
    def msg_kernel(sd_hbm, w_hbm, flo_hbm, fhi_hbm, out_hbm,
                   sd_i, wb, fb, msgb, sem_g, sem_w, out_sp):
        c = lax.axis_index("c")
        s = lax.axis_index("s")
        lane = lax.iota(jnp.int32, 16)
        zi = jnp.zeros((16,), jnp.int32)
        zf = jnp.zeros((16,), jnp.float32)

        for q in range(2):
            iq = jnp.full((16,), q, jnp.int32)

            @pl.loop(0, 128)
            def _(e):
                ie = zi + e
                plsc.store_scatter(msgb, [iq, ie, lane], zf)
                plsc.store_scatter(msgb, [iq, ie, lane + 16], zf)

        @pl.loop(0, 25)
        def _(t):
            g = t * 16 + s

            @pl.when(g < NROWBLK)
            def _():
                pltpu.sync_copy(msgb.at[0], out_sp.at[pl.ds(g * 128, 128)])

        plsc.subcore_barrier()

        ebase = s * (EPT_B // 128)
        wbase = s * EPT_B

        def issue(jj, slot):
            r0 = ebase + jj * 2
            pltpu.sync_copy(sd_hbm.at[pl.ds(r0, 2)], sd_i.at[slot])
            pltpu.async_copy(
                w_hbm.at[pl.ds(wbase + jj * K, K), :], wb.at[slot], sem_w)
            for q in range(2):
                @pl.when(c == 0)
                def _():
                    pltpu.async_copy(
                        flo_hbm.at[sd_i.at[slot, q, 0]], fb.at[slot, q], sem_g)

                @pl.when(c == 1)
                def _():
                    pltpu.async_copy(
                        fhi_hbm.at[sd_i.at[slot, q, 0]], fb.at[slot, q], sem_g)

        def drain(slot):
            pltpu.make_async_copy(
                w_hbm.at[pl.ds(0, K), :], wb.at[slot], sem_w).wait()
            for q in range(2):
                pltpu.make_async_copy(
                    flo_hbm.at[pl.ds(0, 128)], fb.at[slot, q], sem_g).wait()

        issue(0, 0)

        if l1_mode:
            wcol0 = c * 4 + lax.shift_right_logical(lane, 3)
            wcol1 = wcol0 + 2
        else:
            wcol0 = zi
            wcol1 = zi

        @pl.loop(0, NCHUNK_B, step=2)
        def _(j):
            for slot in range(2):
                jj = j + slot
                issue(jnp.minimum(jj + 1, NCHUNK_B - 1), 1 - slot)
                drain(slot)

                for q in range(2):
                    @pl.loop(0, 128, unroll=4)
                    def _(e):
                        iw = zi + (e + q * 128)
                        w0 = plsc.load_gather(wb.at[slot], [iw, wcol0])
                        f0 = fb[slot, q, e, pl.ds(0, 16)]
                        msgb[q, e, pl.ds(0, 16)] = w0 * f0
                        if l1_mode:
                            w1 = plsc.load_gather(wb.at[slot], [iw, wcol1])
                        else:
                            w1 = w0
                        f1 = fb[slot, q, e, pl.ds(16, 16)]
                        msgb[q, e, pl.ds(16, 16)] = w1 * f1

                for q in range(2):
                    pltpu.sync_copy(msgb.at[q],
                                    out_sp.at[sd_i.at[slot, q, 1]], add=True)

        drain(0)
        plsc.subcore_barrier()

        @pl.loop(0, 25)
        def _(t):
            g = t * 16 + s

            @pl.when(g < NROWBLK)
            def _():
                r = g * 128
                pltpu.sync_copy(out_sp.at[pl.ds(r, 128)],
                                out_hbm.at[c, pl.ds(r, 128), :])

    return msg_kernel


_w_l1 = _make_w_kernel(True)
_w_single = _make_w_kernel(False)
_msg_l1 = _make_msg_kernel(True)
_msg_single = _make_msg_kernel(False)


def _tc1_body(x_ref, w1_ref, as_ref, ad_ref, hlo_ref, hhi_ref, atbl_ref):
    h = jnp.dot(x_ref[...], w1_ref[...], preferred_element_type=jnp.float32)
    a_s = jnp.dot(h, as_ref[...], preferred_element_type=jnp.float32)
    a_d = jnp.dot(h, ad_ref[...], preferred_element_type=jnp.float32)
    hlo_ref[...] = h[:, 0:32]
    hhi_ref[...] = h[:, 32:64]
    atbl_ref[...] = jnp.concatenate([a_s, a_d], axis=1)


def _tc2_body(p0_ref, p1_ref, d0_ref, d1_ref, rep_ref, b_ref, w_ref, m_ref,
              flo_ref, fhi_ref, atbl_ref):
    rep = rep_ref[...]
    den = d0_ref[...] + d1_ref[...] + 1e-16
    dd0 = jnp.dot(den[:, 0:4], rep, preferred_element_type=jnp.float32)
    dd1 = jnp.dot(den[:, 4:8], rep, preferred_element_type=jnp.float32)
    x = jnp.concatenate([p0_ref[...] / dd0, p1_ref[...] / dd1],
                        axis=1) + b_ref[...]
    x = _elu(x)
    h = jnp.dot(x, w_ref[...], preferred_element_type=jnp.float32)
    flo_ref[...] = h[:, 0:32]
    fhi_ref[...] = h[:, 32:64]
    atbl_ref[...] = jnp.dot(x, m_ref[...], preferred_element_type=jnp.float32)


def _tc3_body(p0_ref, p1_ref, d0_ref, d1_ref, b_ref, m_ref, flo_ref, fhi_ref,
              atbl_ref):
    d = d0_ref[...][:, 0:1] + d1_ref[...][:, 0:1] + 1e-16
    x = jnp.concatenate([p0_ref[...] / d, p1_ref[...] / d], axis=1) + b_ref[...]
    x = _elu(x)
    flo_ref[...] = x[:, 0:32]
    fhi_ref[...] = x[:, 32:64]
    atbl_ref[...] = jnp.dot(x, m_ref[...], preferred_element_type=jnp.float32)


def _tc4_body(p0_ref, p1_ref, d0_ref, d1_ref, w_ref, b_ref, out_ref):
    d = d0_ref[...][:, 0:1] + d1_ref[...][:, 0:1] + 1e-16
    agg = jnp.concatenate([p0_ref[...] / d, p1_ref[...] / d], axis=1)
    out_ref[...] = jnp.dot(agg, w_ref[...],
                           preferred_element_type=jnp.float32) + b_ref[...]


def _row_spec(c):
    return pl.BlockSpec((BLK, c), lambda i: (i, 0))


def _full_spec(r, c):
    return pl.BlockSpec((r, c), lambda i: (0, 0))


_tc1 = pl.pallas_call(
    _tc1_body,
    grid=(GRID,),
    in_specs=[_row_spec(128), _full_spec(128, 64), _full_spec(64, 8),
              _full_spec(64, 8)],
    out_specs=[_row_spec(32), _row_spec(32), _row_spec(16)],
    out_shape=[jax.ShapeDtypeStruct((NPAD, 32), jnp.float32),
               jax.ShapeDtypeStruct((NPAD, 32), jnp.float32),
               jax.ShapeDtypeStruct((NPAD, 16), jnp.float32)],
)

_tc2 = pl.pallas_call(
    _tc2_body,
    grid=(GRID,),
    in_specs=[_row_spec(32), _row_spec(32), _row_spec(8), _row_spec(8),
              _full_spec(4, 32), _full_spec(1, 64), _full_spec(64, 64),
              _full_spec(64, 16)],
    out_specs=[_row_spec(32), _row_spec(32), _row_spec(16)],
    out_shape=[jax.ShapeDtypeStruct((NPAD, 32), jnp.float32),
               jax.ShapeDtypeStruct((NPAD, 32), jnp.float32),
               jax.ShapeDtypeStruct((NPAD, 16), jnp.float32)],
)

_tc3 = pl.pallas_call(
    _tc3_body,
    grid=(GRID,),
    in_specs=[_row_spec(32), _row_spec(32), _row_spec(8), _row_spec(8),
              _full_spec(1, 64), _full_spec(64, 16)],
    out_specs=[_row_spec(32), _row_spec(32), _row_spec(16)],
    out_shape=[jax.ShapeDtypeStruct((NPAD, 32), jnp.float32),
               jax.ShapeDtypeStruct((NPAD, 32), jnp.float32),
               jax.ShapeDtypeStruct((NPAD, 16), jnp.float32)],
)

_tc4 = pl.pallas_call(
    _tc4_body,
    grid=(GRID,),
    in_specs=[_row_spec(32), _row_spec(32), _row_spec(8), _row_spec(8),
              _full_spec(64, 128), _full_spec(1, 128)],
    out_specs=_row_spec(128),
    out_shape=jax.ShapeDtypeStruct((NPAD, 128), jnp.float32),
)


def kernel(x, edge_index, W1, a_src1, a_dst1, b1, W2, a_src2, a_dst2, b2,
           W3, a_src3, a_dst3, b3):
    f32 = jnp.float32
    xp = jnp.pad(x.astype(f32), ((0, NPAD - N_NODES), (0, 128 - x.shape[1])))
    w1p = jnp.pad(W1, ((0, 128 - W1.shape[0]), (0, 0)))
    eye8 = jnp.eye(8, dtype=f32)
    as1 = (a_src1[0][:, :, None] * eye8[:, None, :]).reshape(64, 8)
    ad1 = (a_dst1[0][:, :, None] * eye8[:, None, :]).reshape(64, 8)
    m2 = jnp.zeros((64, 16), f32)
    m2 = m2.at[:, 0].set(W2 @ a_src2[0, 0]).at[:, 1].set(W2 @ a_dst2[0, 0])
    m3 = jnp.zeros((64, 16), f32)
    m3 = m3.at[:, 0].set(W3 @ a_src3[0, 0]).at[:, 1].set(W3 @ a_dst3[0, 0])
    rep = jnp.repeat(jnp.eye(4, dtype=f32), 8, axis=1)
    w3p = jnp.pad(W3, ((0, 0), (0, 128 - W3.shape[1])))
    b3p = jnp.pad(b3, (0, 128 - b3.shape[0])).reshape(1, 128)
    b1r = b1.reshape(1, 64)
    b2r = b2.reshape(1, 64)
    src = jnp.concatenate(
        [edge_index[0], jnp.zeros((EPAD - N_EDGES,), jnp.int32)])
    dst = jnp.concatenate(
        [edge_index[1], jnp.full((EPAD - N_EDGES,), DUMP_ROW, jnp.int32)])
    sd2d = jnp.stack(
        [src.reshape(EPAD // 128, 128), dst.reshape(EPAD // 128, 128)],
        axis=1)

    h1lo, h1hi, atbl1 = _tc1(xp, w1p, as1, ad1)
    w1e, den1 = _w_l1(sd2d, atbl1)
    out1 = _msg_l1(sd2d, w1e, h1lo, h1hi)
    f2lo, f2hi, atbl2 = _tc2(out1[0], out1[1], den1[0], den1[1], rep, b1r,
                             W2, m2)
    w2e, den2 = _w_single(sd2d, atbl2)
    out2 = _msg_single(sd2d, w2e, f2lo, f2hi)
    f3lo, f3hi, atbl3 = _tc3(out2[0], out2[1], den2[0], den2[1], b2r, m3)
    w3e, den3 = _w_single(sd2d, atbl3)
    out3 = _msg_single(sd2d, w3e, f3lo, f3hi)
    res = _tc4(out3[0], out3[1], den3[0], den3[1], w3p, b3p)
    return res[:N_NODES, :121]
